# Initial kernel scaffold; baseline (speedup 1.0000x reference)
import jax
import jax.numpy as jnp
from jax.experimental import pallas as pl


def kernel(features, edge_index, transport_angles):
    raise NotImplementedError("write your pallas kernel here")


# speedup vs baseline: 8.2010x; 8.2010x over previous
import dataclasses
import functools

import jax
import jax.numpy as jnp
from jax import lax
from jax.experimental import pallas as pl
from jax.experimental.pallas import tpu as pltpu
from jax.experimental.pallas import tpu_sc as plsc

NC = 2
NS = 16
NW = NC * NS
L = 16

CHUNK = 1000


def _trig_kernel(a_ref, c_ref, s_ref):
    x = a_ref[...]
    c_ref[...] = jnp.cos(x)
    s_ref[...] = jnp.sin(x)


def _compute_trig(angles2d):
    c, s = pl.pallas_call(
        _trig_kernel,
        out_shape=(
            jax.ShapeDtypeStruct(angles2d.shape, jnp.float32),
            jax.ShapeDtypeStruct(angles2d.shape, jnp.float32),
        ),
    )(angles2d)
    return c, s


def _sc_gather_rotate(table, idx, c, s):
    E = idx.shape[0]
    D = table.shape[1]
    per_w = E // NW
    mesh = plsc.VectorSubcoreMesh(core_axis_name="c", subcore_axis_name="s")
    cp = pltpu.CompilerParams(use_tc_tiling_on_sc=False)
    if "needs_layout_passes" in pltpu.CompilerParams.__dataclass_fields__:
        cp = dataclasses.replace(cp, needs_layout_passes=False)

    @functools.partial(
        pl.kernel,
        mesh=mesh,
        compiler_params=cp,
        out_type=jax.ShapeDtypeStruct((E, D), jnp.float32),
        scratch_types=[
            pltpu.VMEM((CHUNK,), jnp.int32),
            pltpu.VMEM((CHUNK,), jnp.float32),
            pltpu.VMEM((CHUNK,), jnp.float32),
            pltpu.VMEM((CHUNK, D), jnp.float32),
            pltpu.VMEM((CHUNK, D), jnp.float32),
            pltpu.SemaphoreType.DMA,
        ],
    )
    def k(table_hbm, idx_hbm, c_hbm, s_hbm, out_hbm,
          idx_v, c_v, s_v, rows_v, out_v, sem):
        wid = lax.axis_index("s") * NC + lax.axis_index("c")

        @pl.loop(0, per_w, step=CHUNK)
        def _(off):
            base = wid * per_w + off
            pltpu.sync_copy(idx_hbm.at[pl.ds(base, CHUNK)], idx_v)
            pltpu.sync_copy(c_hbm.at[pl.ds(base, CHUNK)], c_v)
            pltpu.sync_copy(s_hbm.at[pl.ds(base, CHUNK)], s_v)
            pltpu.async_copy(table_hbm.at[idx_v], rows_v, sem).wait()

            @pl.loop(0, CHUNK, step=L)
            def _(g):
                cvec = c_v[pl.ds(g, L)]
                svec = s_v[pl.ds(g, L)]
                rvec = lax.iota(jnp.int32, L) + g
                for j in range(D // 2):
                    jx = lax.broadcast(jnp.int32(2 * j), (L,))
                    jy = lax.broadcast(jnp.int32(2 * j + 1), (L,))
                    x = plsc.load_gather(rows_v, [rvec, jx])
                    y = plsc.load_gather(rows_v, [rvec, jy])
                    plsc.store_scatter(out_v, [rvec, jx], cvec * x - svec * y)
                    plsc.store_scatter(out_v, [rvec, jy], svec * x + cvec * y)

            pltpu.sync_copy(out_v, out_hbm.at[pl.ds(base, CHUNK)])

    return k(table, idx, c, s)


def kernel(features, edge_index, transport_angles):
    B, N, C, two = features.shape
    E = edge_index.shape[1]
    table = features.reshape(N, C * two)
    row = edge_index[0]
    angles2d = transport_angles.reshape(E // 128, 128)
    c, s = _compute_trig(angles2d)
    out = _sc_gather_rotate(table, row, c.reshape(E), s.reshape(E))
    return out.reshape(B, E, C, two)


# speedup vs baseline: 8.5111x; 1.0378x over previous
import dataclasses
import functools

import jax
import jax.numpy as jnp
from jax import lax
from jax.experimental import pallas as pl
from jax.experimental.pallas import tpu as pltpu
from jax.experimental.pallas import tpu_sc as plsc

NC = 2
NS = 16
NW = NC * NS
L = 16

CHUNK = 2000


def _trig_kernel(a_ref, c_ref, s_ref):
    x = a_ref[...]
    c_ref[...] = jnp.cos(x)
    s_ref[...] = jnp.sin(x)


def _compute_trig(angles2d):
    c, s = pl.pallas_call(
        _trig_kernel,
        out_shape=(
            jax.ShapeDtypeStruct(angles2d.shape, jnp.float32),
            jax.ShapeDtypeStruct(angles2d.shape, jnp.float32),
        ),
    )(angles2d)
    return c, s


def _sc_gather_rotate(table, idx, c, s):
    E = idx.shape[0]
    D = table.shape[1]
    per_w = E // NW
    mesh = plsc.VectorSubcoreMesh(core_axis_name="c", subcore_axis_name="s")
    cp = pltpu.CompilerParams(use_tc_tiling_on_sc=False)
    if "needs_layout_passes" in pltpu.CompilerParams.__dataclass_fields__:
        cp = dataclasses.replace(cp, needs_layout_passes=False)

    @functools.partial(
        pl.kernel,
        mesh=mesh,
        compiler_params=cp,
        out_type=jax.ShapeDtypeStruct((E, D), jnp.float32),
        scratch_types=[
            pltpu.VMEM((CHUNK,), jnp.int32),
            pltpu.VMEM((CHUNK,), jnp.float32),
            pltpu.VMEM((CHUNK,), jnp.float32),
            pltpu.VMEM((CHUNK, D), jnp.float32),
            pltpu.VMEM((CHUNK, D), jnp.float32),
            pltpu.SemaphoreType.DMA,
        ],
    )
    def k(table_hbm, idx_hbm, c_hbm, s_hbm, out_hbm,
          idx_v, c_v, s_v, rows_v, out_v, sem):
        wid = lax.axis_index("s") * NC + lax.axis_index("c")

        @pl.loop(0, per_w, step=CHUNK)
        def _(off):
            base = wid * per_w + off
            pltpu.sync_copy(idx_hbm.at[pl.ds(base, CHUNK)], idx_v)
            pltpu.sync_copy(c_hbm.at[pl.ds(base, CHUNK)], c_v)
            pltpu.sync_copy(s_hbm.at[pl.ds(base, CHUNK)], s_v)
            pltpu.async_copy(table_hbm.at[idx_v], rows_v, sem).wait()

            @pl.loop(0, CHUNK, step=L)
            def _(g):
                cvec = c_v[pl.ds(g, L)]
                svec = s_v[pl.ds(g, L)]
                rvec = lax.iota(jnp.int32, L) + g
                for j in range(D // 2):
                    jx = lax.broadcast(jnp.int32(2 * j), (L,))
                    jy = lax.broadcast(jnp.int32(2 * j + 1), (L,))
                    x = plsc.load_gather(rows_v, [rvec, jx])
                    y = plsc.load_gather(rows_v, [rvec, jy])
                    plsc.store_scatter(out_v, [rvec, jx], cvec * x - svec * y)
                    plsc.store_scatter(out_v, [rvec, jy], svec * x + cvec * y)

            pltpu.sync_copy(out_v, out_hbm.at[pl.ds(base, CHUNK)])

    return k(table, idx, c, s)


def kernel(features, edge_index, transport_angles):
    B, N, C, two = features.shape
    E = edge_index.shape[1]
    table = features.reshape(N, C * two)
    row = edge_index[0]
    angles2d = transport_angles.reshape(E // 128, 128)
    c, s = _compute_trig(angles2d)
    out = _sc_gather_rotate(table, row, c.reshape(E), s.reshape(E))
    return out.reshape(B, E, C, two)
